# baseline (device time: 57079 ns/iter reference)
import os

import jax
import jax.numpy as jnp
from jax import lax
from jax.experimental import pallas as pl
from jax.experimental.pallas import tpu as pltpu

_VARIANT = os.environ.get("KERNEL_VARIANT", "full")

N = 16
SQ = 256
D = 1024
SKV = 4096
NH = 8
DH = 128
ROWS = SQ // N
SCALE = 0.08838834764831843

_NN = (((1,), (0,)), ((), ()))
_NT = (((1,), (1,)), ((), ()))


def _rs_start(me, part_ref, comm_ref, send_sems, recv_sems):
    for peer in range(N):
        @pl.when(peer != me)
        def _():
            pltpu.make_async_remote_copy(
                src_ref=part_ref.at[pl.ds(peer * ROWS, ROWS), :],
                dst_ref=comm_ref.at[me],
                send_sem=send_sems.at[peer],
                recv_sem=recv_sems.at[me],
                device_id=(peer,),
                device_id_type=pl.DeviceIdType.MESH,
            ).start()


def _rs_wait_recv(me, comm_ref, send_sems, recv_sems):
    for src in range(N):
        @pl.when(src != me)
        def _():
            pltpu.make_async_remote_copy(
                src_ref=comm_ref.at[src],
                dst_ref=comm_ref.at[src],
                send_sem=send_sems.at[src],
                recv_sem=recv_sems.at[src],
                device_id=(src,),
                device_id_type=pl.DeviceIdType.MESH,
            ).wait_recv()


def _rs_wait_send(me, part_ref, comm_ref, send_sems, recv_sems):
    for peer in range(N):
        @pl.when(peer != me)
        def _():
            pltpu.make_async_remote_copy(
                src_ref=part_ref.at[pl.ds(peer * ROWS, ROWS), :],
                dst_ref=comm_ref.at[me],
                send_sem=send_sems.at[peer],
                recv_sem=recv_sems.at[me],
                device_id=(peer,),
                device_id_type=pl.DeviceIdType.MESH,
            ).wait_send()


def _body(x_ref, wq_ref, wo_ref, k_ref, v_ref, out_ref,
          kbuf, vbuf, kv_sems, part_a, part_b, comm_a, comm_b,
          rsa_send, rsa_recv, rsb_send, rsb_recv, ag_send, ag_recv):
    me = lax.axis_index("i")
    bf16 = jnp.bfloat16
    f32 = jnp.float32

    def kv_copy(h, slot):
        return (
            pltpu.make_async_copy(
                k_ref.at[0, :, h, :], kbuf.at[slot], kv_sems.at[0, slot]),
            pltpu.make_async_copy(
                v_ref.at[0, :, h, :], vbuf.at[slot], kv_sems.at[1, slot]),
        )

    for c in kv_copy(0, 0):
        c.start()

    q = lax.dot_general(x_ref[...], wq_ref[...], _NN,
                        preferred_element_type=f32)
    q = (q * SCALE).astype(bf16)

    ones = jnp.ones((SKV, 128), bf16)
    part = None
    for h in range(NH):
        if h + 1 < NH:
            for c in kv_copy(h + 1, (h + 1) % 2):
                c.start()
        for c in kv_copy(h, h % 2):
            c.wait()
        qh = q[:, h * DH:(h + 1) * DH]
        kh = kbuf[h % 2].astype(bf16)
        vh = vbuf[h % 2].astype(bf16)
        s = lax.dot_general(qh, kh, _NT, preferred_element_type=f32)
        p = jnp.exp(s.astype(bf16))
        l = lax.dot_general(p, ones, _NN, preferred_element_type=f32)[:, :1]
        o = lax.dot_general(p, vh, _NN, preferred_element_type=f32)
        o = (o / l).astype(bf16)
        contrib = lax.dot_general(
            o, wo_ref[h * DH:(h + 1) * DH, :], _NN, preferred_element_type=f32)
        part = contrib if part is None else part + contrib
        if h == NH // 2 - 1 and _VARIANT == "full":
            part_a[...] = part.astype(bf16)
            _rs_start(me, part_a, comm_a, rsa_send, rsa_recv)
            part = None
    part_b[...] = part.astype(bf16)

    if _VARIANT == "compute_only":
        out_ref[...] = part_b[...]
        return

    _rs_start(me, part_b, comm_b, rsb_send, rsb_recv)

    comm_a[pl.ds(me, 1)] = part_a[pl.ds(me * ROWS, ROWS), :][None]
    comm_b[pl.ds(me, 1)] = part_b[pl.ds(me * ROWS, ROWS), :][None]

    _rs_wait_recv(me, comm_a, rsa_send, rsa_recv)
    _rs_wait_recv(me, comm_b, rsb_send, rsb_recv)

    acc = comm_a[0].astype(f32) + comm_b[0].astype(f32)
    for s_ in range(1, N):
        acc = acc + comm_a[s_].astype(f32) + comm_b[s_].astype(f32)
    out_ref[pl.ds(me * ROWS, ROWS), :] = acc.astype(bf16)

    for peer in range(N):
        @pl.when(peer != me)
        def _():
            pltpu.make_async_remote_copy(
                src_ref=out_ref.at[pl.ds(me * ROWS, ROWS), :],
                dst_ref=out_ref.at[pl.ds(me * ROWS, ROWS), :],
                send_sem=ag_send.at[peer],
                recv_sem=ag_recv.at[me],
                device_id=(peer,),
                device_id_type=pl.DeviceIdType.MESH,
            ).start()

    for src in range(N):
        @pl.when(src != me)
        def _():
            pltpu.make_async_remote_copy(
                src_ref=out_ref.at[pl.ds(src * ROWS, ROWS), :],
                dst_ref=out_ref.at[pl.ds(src * ROWS, ROWS), :],
                send_sem=ag_send.at[src],
                recv_sem=ag_recv.at[src],
                device_id=(src,),
                device_id_type=pl.DeviceIdType.MESH,
            ).wait_recv()

    _rs_wait_send(me, part_a, comm_a, rsa_send, rsa_recv)
    _rs_wait_send(me, part_b, comm_b, rsb_send, rsb_recv)
    for peer in range(N):
        @pl.when(peer != me)
        def _():
            pltpu.make_async_remote_copy(
                src_ref=out_ref.at[pl.ds(me * ROWS, ROWS), :],
                dst_ref=out_ref.at[pl.ds(me * ROWS, ROWS), :],
                send_sem=ag_send.at[peer],
                recv_sem=ag_recv.at[me],
                device_id=(peer,),
                device_id_type=pl.DeviceIdType.MESH,
            ).wait_send()


def kernel(x, Wq, Wo, K_ext, V_ext):
    bf16 = jnp.bfloat16
    out = pl.pallas_call(
        _body,
        out_shape=jax.ShapeDtypeStruct((SQ, D), bf16),
        in_specs=[
            pl.BlockSpec(memory_space=pltpu.VMEM),
            pl.BlockSpec(memory_space=pltpu.VMEM),
            pl.BlockSpec(memory_space=pltpu.VMEM),
            pl.BlockSpec(memory_space=pl.ANY),
            pl.BlockSpec(memory_space=pl.ANY),
        ],
        out_specs=pl.BlockSpec(memory_space=pltpu.VMEM),
        scratch_shapes=[
            pltpu.VMEM((2, SKV, DH), jnp.float32),
            pltpu.VMEM((2, SKV, DH), jnp.float32),
            pltpu.SemaphoreType.DMA((2, 2)),
            pltpu.VMEM((SQ, D), bf16),
            pltpu.VMEM((SQ, D), bf16),
            pltpu.VMEM((N, ROWS, D), bf16),
            pltpu.VMEM((N, ROWS, D), bf16),
            pltpu.SemaphoreType.DMA((N,)),
            pltpu.SemaphoreType.DMA((N,)),
            pltpu.SemaphoreType.DMA((N,)),
            pltpu.SemaphoreType.DMA((N,)),
            pltpu.SemaphoreType.DMA((N,)),
            pltpu.SemaphoreType.DMA((N,)),
        ],
        compiler_params=pltpu.CompilerParams(
            vmem_limit_bytes=128 * 1024 * 1024,
        ),
    )(
        x.reshape(SQ, D).astype(bf16),
        Wq.astype(bf16),
        Wo.astype(bf16),
        K_ext,
        V_ext,
    )
    return out.reshape(1, SQ, D)


# device time: 40149 ns/iter; 1.4217x vs baseline; 1.4217x over previous
import os

import jax
import jax.numpy as jnp
from jax import lax
from jax.experimental import pallas as pl
from jax.experimental.pallas import tpu as pltpu

_VARIANT = os.environ.get("KERNEL_VARIANT", "full")
_SPLIT = os.environ.get("KERNEL_SPLIT", "none")

N = 16
SQ = 256
D = 1024
SKV = 4096
NH = 8
DH = 128
ROWS = SQ // N
SCALE = 0.08838834764831843

_NN = (((1,), (0,)), ((), ()))
_NT = (((1,), (1,)), ((), ()))


def _rs_start(me, part_ref, comm_ref, send_sems, recv_sems):
    for peer in range(N):
        @pl.when(peer != me)
        def _():
            pltpu.make_async_remote_copy(
                src_ref=part_ref.at[pl.ds(peer * ROWS, ROWS), :],
                dst_ref=comm_ref.at[me],
                send_sem=send_sems.at[peer],
                recv_sem=recv_sems.at[me],
                device_id=(peer,),
                device_id_type=pl.DeviceIdType.MESH,
            ).start()


def _rs_wait_recv(me, comm_ref, send_sems, recv_sems):
    for src in range(N):
        @pl.when(src != me)
        def _():
            pltpu.make_async_remote_copy(
                src_ref=comm_ref.at[src],
                dst_ref=comm_ref.at[src],
                send_sem=send_sems.at[src],
                recv_sem=recv_sems.at[src],
                device_id=(src,),
                device_id_type=pl.DeviceIdType.MESH,
            ).wait_recv()


def _rs_wait_send(me, part_ref, comm_ref, send_sems, recv_sems):
    for peer in range(N):
        @pl.when(peer != me)
        def _():
            pltpu.make_async_remote_copy(
                src_ref=part_ref.at[pl.ds(peer * ROWS, ROWS), :],
                dst_ref=comm_ref.at[me],
                send_sem=send_sems.at[peer],
                recv_sem=recv_sems.at[me],
                device_id=(peer,),
                device_id_type=pl.DeviceIdType.MESH,
            ).wait_send()


def _body(x_ref, wq_ref, wo_ref, k_ref, v_ref, out_ref,
          kbuf, vbuf, kv_sems, part_a, part_b, comm_a, comm_b,
          rsa_send, rsa_recv, rsb_send, rsb_recv, ag_send, ag_recv):
    me = lax.axis_index("i")
    bf16 = jnp.bfloat16
    f32 = jnp.float32

    def kv_copy(h, slot):
        return (
            pltpu.make_async_copy(
                k_ref.at[0, :, h, :], kbuf.at[slot], kv_sems.at[0, slot]),
            pltpu.make_async_copy(
                v_ref.at[0, :, h, :], vbuf.at[slot], kv_sems.at[1, slot]),
        )

    for c in kv_copy(0, 0):
        c.start()

    if _VARIANT == "full":
        barrier_sem = pltpu.get_barrier_semaphore()
        for peer in range(N):
            @pl.when(peer != me)
            def _():
                pl.semaphore_signal(
                    barrier_sem, inc=1,
                    device_id=(peer,),
                    device_id_type=pl.DeviceIdType.MESH,
                )

    q = lax.dot_general(x_ref[...].astype(bf16), wq_ref[...].astype(bf16),
                        _NN, preferred_element_type=f32)
    q = (q * SCALE).astype(bf16)
    wo = wo_ref[...].astype(bf16)

    part = None
    for h in range(NH):
        if h + 1 < NH:
            for c in kv_copy(h + 1, (h + 1) % 2):
                c.start()
        for c in kv_copy(h, h % 2):
            c.wait()
        qh = q[:, h * DH:(h + 1) * DH]
        kh = kbuf[h % 2].astype(bf16)
        vh = vbuf[h % 2].astype(bf16)
        s = lax.dot_general(qh, kh, _NT, preferred_element_type=f32)
        p = jnp.exp(s)
        l = jnp.sum(p, axis=1, keepdims=True)
        o = lax.dot_general(p.astype(bf16), vh, _NN, preferred_element_type=f32)
        o = (o / l).astype(bf16)
        contrib = lax.dot_general(
            o, wo[h * DH:(h + 1) * DH, :], _NN, preferred_element_type=f32)
        part = contrib if part is None else part + contrib
        if h == NH // 2 - 1 and _VARIANT == "full" and _SPLIT == "half":
            part_a[...] = part.astype(bf16)
            pl.semaphore_wait(barrier_sem, N - 1)
            _rs_start(me, part_a, comm_a, rsa_send, rsa_recv)
            part = None
    part_b[...] = part.astype(bf16)

    if _VARIANT == "compute_only":
        out_ref[...] = part_b[...]
        return

    if _SPLIT == "none":
        pl.semaphore_wait(barrier_sem, N - 1)
    _rs_start(me, part_b, comm_b, rsb_send, rsb_recv)

    if _SPLIT == "half":
        comm_a[pl.ds(me, 1)] = part_a[pl.ds(me * ROWS, ROWS), :][None]
    comm_b[pl.ds(me, 1)] = part_b[pl.ds(me * ROWS, ROWS), :][None]

    if _SPLIT == "half":
        _rs_wait_recv(me, comm_a, rsa_send, rsa_recv)
    _rs_wait_recv(me, comm_b, rsb_send, rsb_recv)

    acc = comm_b[0].astype(f32)
    for s_ in range(1, N):
        acc = acc + comm_b[s_].astype(f32)
    if _SPLIT == "half":
        for s_ in range(N):
            acc = acc + comm_a[s_].astype(f32)
    out_ref[pl.ds(me * ROWS, ROWS), :] = acc.astype(bf16)

    for peer in range(N):
        @pl.when(peer != me)
        def _():
            pltpu.make_async_remote_copy(
                src_ref=out_ref.at[pl.ds(me * ROWS, ROWS), :],
                dst_ref=out_ref.at[pl.ds(me * ROWS, ROWS), :],
                send_sem=ag_send.at[peer],
                recv_sem=ag_recv.at[me],
                device_id=(peer,),
                device_id_type=pl.DeviceIdType.MESH,
            ).start()

    for src in range(N):
        @pl.when(src != me)
        def _():
            pltpu.make_async_remote_copy(
                src_ref=out_ref.at[pl.ds(src * ROWS, ROWS), :],
                dst_ref=out_ref.at[pl.ds(src * ROWS, ROWS), :],
                send_sem=ag_send.at[src],
                recv_sem=ag_recv.at[src],
                device_id=(src,),
                device_id_type=pl.DeviceIdType.MESH,
            ).wait_recv()

    if _SPLIT == "half":
        _rs_wait_send(me, part_a, comm_a, rsa_send, rsa_recv)
    _rs_wait_send(me, part_b, comm_b, rsb_send, rsb_recv)
    for peer in range(N):
        @pl.when(peer != me)
        def _():
            pltpu.make_async_remote_copy(
                src_ref=out_ref.at[pl.ds(me * ROWS, ROWS), :],
                dst_ref=out_ref.at[pl.ds(me * ROWS, ROWS), :],
                send_sem=ag_send.at[peer],
                recv_sem=ag_recv.at[me],
                device_id=(peer,),
                device_id_type=pl.DeviceIdType.MESH,
            ).wait_send()


def kernel(x, Wq, Wo, K_ext, V_ext):
    bf16 = jnp.bfloat16
    out = pl.pallas_call(
        _body,
        out_shape=jax.ShapeDtypeStruct((SQ, D), bf16),
        in_specs=[
            pl.BlockSpec(memory_space=pltpu.VMEM),
            pl.BlockSpec(memory_space=pltpu.VMEM),
            pl.BlockSpec(memory_space=pltpu.VMEM),
            pl.BlockSpec(memory_space=pl.ANY),
            pl.BlockSpec(memory_space=pl.ANY),
        ],
        out_specs=pl.BlockSpec(memory_space=pltpu.VMEM),
        scratch_shapes=[
            pltpu.VMEM((2, SKV, DH), jnp.float32),
            pltpu.VMEM((2, SKV, DH), jnp.float32),
            pltpu.SemaphoreType.DMA((2, 2)),
            pltpu.VMEM((SQ, D), bf16),
            pltpu.VMEM((SQ, D), bf16),
            pltpu.VMEM((N, ROWS, D), bf16),
            pltpu.VMEM((N, ROWS, D), bf16),
            pltpu.SemaphoreType.DMA((N,)),
            pltpu.SemaphoreType.DMA((N,)),
            pltpu.SemaphoreType.DMA((N,)),
            pltpu.SemaphoreType.DMA((N,)),
            pltpu.SemaphoreType.DMA((N,)),
            pltpu.SemaphoreType.DMA((N,)),
        ],
        compiler_params=pltpu.CompilerParams(
            vmem_limit_bytes=128 * 1024 * 1024,
            **({"collective_id": 0} if _VARIANT == "full" else {}),
        ),
    )(
        x.reshape(SQ, D),
        Wq,
        Wo,
        K_ext,
        V_ext,
    )
    return out.reshape(1, SQ, D)
